# Initial kernel scaffold; baseline (speedup 1.0000x reference)
import jax
import jax.numpy as jnp
from jax.experimental import pallas as pl


def kernel(x, table):
    raise NotImplementedError("write your pallas kernel here")


# speedup vs baseline: 2.5264x; 2.5264x over previous
import jax
import jax.numpy as jnp
from jax.experimental import pallas as pl


def _body(x_ref, t_ref, o_ref):
    o_ref[...] = x_ref[...] + t_ref[...]


def kernel(x, table):
    B, S, D = x.shape
    bs = 512
    return pl.pallas_call(
        _body,
        grid=(B, S // bs),
        in_specs=[
            pl.BlockSpec((1, bs, D), lambda b, i: (b, i, 0)),
            pl.BlockSpec((bs, D), lambda b, i: (i, 0)),
        ],
        out_specs=pl.BlockSpec((1, bs, D), lambda b, i: (b, i, 0)),
        out_shape=jax.ShapeDtypeStruct((B, S, D), x.dtype),
    )(x, table)
